# baseline (speedup 1.0000x reference)
import functools

import jax
import jax.numpy as jnp
from jax import lax
from jax.experimental import pallas as pl
from jax.experimental.pallas import tpu as pltpu
from jax.experimental.pallas import tpu_sc as plsc

_NB = 16
_RADIUS = 0.05
_BLOCK = 1000

_N = 100000
_NW = 32
_A = _N // _NW
_CH = 120
_STAGE = 3136
_NPAD = 100096


def _tc_body(f_ref, p_ref, w_ref, br_ref, out_ref):
    f = f_ref[...].astype(jnp.bfloat16)
    rel = jnp.dot(f, w_ref[...], preferred_element_type=jnp.float32)
    p = p_ref[...]
    anchor = jnp.concatenate([p] * _NB, axis=1)
    out_ref[:, : _NB * 3] = anchor + rel + br_ref[...]


def _sc_points_body(rows_ref, x_ref, inv_ref, outv_ref):
    wid = lax.axis_index("s") * 2 + lax.axis_index("c")
    s = (wid * _A) // 8 * 8
    e = ((wid + 1) * _A) // 8 * 8
    cols = [3 * lax.iota(jnp.int32, 16) + j for j in range(3)]

    def do_span(a0, nch, ch):
        def chunk(c, carry):
            ac = a0 + c * ch
            pltpu.sync_copy(rows_ref.at[pl.ds(ac, ch), :],
                            inv_ref.at[pl.ds(0, ch), :])

            def group(g, carry2):
                for u in range(8):
                    a = g * 8 + u
                    row = jnp.zeros((16,), jnp.int32) + a
                    for j in range(3):
                        v = plsc.load_gather(inv_ref, [row, cols[j]])
                        outv_ref[j, pl.ds(a * 16, 16)] = v
                return carry2

            lax.fori_loop(0, ch // 8, group, 0)
            pltpu.sync_copy(outv_ref.at[:, pl.ds(0, ch * 16)],
                            x_ref.at[:, pl.ds(ac * 16, ch * 16)])
            return carry

        lax.fori_loop(0, nch, chunk, 0)

    nfull = (e - s) // _CH
    do_span(s, nfull, _CH)
    @pl.when(e - s - nfull * _CH == 8)
    def _():
        do_span(s + nfull * _CH, 1, 8)


def _sc_batch_body(batch_ref, outb_ref, stage_ref, outbv_ref):
    wid = lax.axis_index("s") * 2 + lax.axis_index("c")
    base = wid * _A
    astart = (base // 8) * 8
    off = base - astart
    pltpu.sync_copy(batch_ref.at[pl.ds(astart, _STAGE)], stage_ref)

    def bgroup(g, carry):
        for u in range(5):
            t = g * 5 + u
            idx = jnp.zeros((16,), jnp.int32) + (t + off)
            outbv_ref[pl.ds(t * 16, 16)] = plsc.load_gather(stage_ref, [idx])
        return carry

    lax.fori_loop(0, _A // 5, bgroup, 0)
    pltpu.sync_copy(outbv_ref, outb_ref.at[pl.ds(base * _NB, _A * _NB)])


def kernel(points, features, batch, W, b):
    n, d = features.shape
    wr = (W * _RADIUS).astype(jnp.bfloat16)
    br = (b * _RADIUS).reshape(1, _NB * 3)

    batch_padded = jnp.pad(batch, (0, _NPAD - n))
    mesh = plsc.VectorSubcoreMesh(core_axis_name="c", subcore_axis_name="s")
    out_batch = functools.partial(
        pl.kernel,
        out_type=jax.ShapeDtypeStruct((n * _NB,), batch.dtype),
        mesh=mesh,
        compiler_params=pltpu.CompilerParams(needs_layout_passes=False),
        scratch_types=[
            pltpu.VMEM((_STAGE,), jnp.int32),
            pltpu.VMEM((_A * _NB,), jnp.int32),
        ],
    )(_sc_batch_body)(batch_padded)

    rows = pl.pallas_call(
        _tc_body,
        grid=(n // _BLOCK,),
        in_specs=[
            pl.BlockSpec((_BLOCK, d), lambda i: (i, 0)),
            pl.BlockSpec((_BLOCK, 3), lambda i: (i, 0)),
            pl.BlockSpec((d, _NB * 3), lambda i: (0, 0)),
            pl.BlockSpec((1, _NB * 3), lambda i: (0, 0)),
        ],
        out_specs=pl.BlockSpec((_BLOCK, 128), lambda i: (i, 0)),
        out_shape=jax.ShapeDtypeStruct((n, 128), jnp.float32),
    )(features, points, wr, br)

    xt = functools.partial(
        pl.kernel,
        out_type=jax.ShapeDtypeStruct((3, n * _NB), jnp.float32),
        mesh=mesh,
        compiler_params=pltpu.CompilerParams(needs_layout_passes=False),
        scratch_types=[
            pltpu.VMEM((_CH, 128), jnp.float32),
            pltpu.VMEM((3, _CH * _NB), jnp.float32),
        ],
    )(_sc_points_body)(rows)

    return xt.T, out_batch


# speedup vs baseline: 1.0909x; 1.0909x over previous
import functools

import jax
import jax.numpy as jnp
from jax import lax
from jax.experimental import pallas as pl
from jax.experimental.pallas import tpu as pltpu
from jax.experimental.pallas import tpu_sc as plsc

_NB = 16
_RADIUS = 0.05
_BLOCK = 2000

_N = 100000
_NW = 32
_A = _N // _NW
_CH = 120
_NCH = 26
_STAGE = 3136


def _tc_body(f_ref, p_ref, w_ref, br_ref, out_ref):
    f = f_ref[...].astype(jnp.bfloat16)
    rel = jnp.dot(f, w_ref[...], preferred_element_type=jnp.float32)
    p = p_ref[...]
    anchor = jnp.concatenate([p] * _NB, axis=1)
    out_ref[:, : _NB * 3] = anchor + rel + br_ref[...]


def _sc_points_body(rows_ref, x_ref, inv0_ref, inv1_ref, outv_ref,
                    sem0_ref, sem1_ref):
    wid = lax.axis_index("s") * 2 + lax.axis_index("c")
    s = (wid * _A) // 8 * 8
    e = ((wid + 1) * _A) // 8 * 8
    cols = [3 * lax.iota(jnp.int32, 16) + j for j in range(3)]
    invs = [inv0_ref, inv1_ref]
    sems = [sem0_ref, sem1_ref]

    def in_copy(c, buf):
        return pltpu.make_async_copy(
            rows_ref.at[pl.ds(s + c * _CH, _CH), :], invs[buf], sems[buf])

    def process(ac, inv_ref, ch):
        def group(g, carry):
            for u in range(8):
                a = g * 8 + u
                row = jnp.zeros((16,), jnp.int32) + a
                for j in range(3):
                    v = plsc.load_gather(inv_ref, [row, cols[j]])
                    outv_ref[j, pl.ds(a * 16, 16)] = v
            return carry

        lax.fori_loop(0, ch // 8, group, 0)
        pltpu.sync_copy(outv_ref.at[:, pl.ds(0, ch * 16)],
                        x_ref.at[:, pl.ds(ac * 16, ch * 16)])

    in_copy(0, 0).start()
    in_copy(1, 1).start()

    def pair(p_idx, carry):
        for b in range(2):
            c = p_idx * 2 + b
            in_copy(c, b).wait()

            @pl.when(c + 2 < _NCH)
            def _():
                in_copy(c + 2, b).start()

            process(s + c * _CH, invs[b], _CH)
        return carry

    lax.fori_loop(0, _NCH // 2, pair, 0)

    @pl.when(e - s - _NCH * _CH == 8)
    def _():
        a0 = s + _NCH * _CH
        pltpu.sync_copy(rows_ref.at[pl.ds(a0, 8), :],
                        inv0_ref.at[pl.ds(0, 8), :])
        process(a0, inv0_ref, 8)


def _sc_batch_body(batch_ref, outb_ref, stage_ref, outbv_ref):
    wid = lax.axis_index("s") * 2 + lax.axis_index("c")
    base = wid * _A
    astart = jnp.minimum((base // 8) * 8, _N - _STAGE)
    off = base - astart
    pltpu.sync_copy(batch_ref.at[pl.ds(astart, _STAGE)], stage_ref)

    def bgroup(g, carry):
        for u in range(5):
            t = g * 5 + u
            idx = jnp.zeros((16,), jnp.int32) + (t + off)
            outbv_ref[pl.ds(t * 16, 16)] = plsc.load_gather(stage_ref, [idx])
        return carry

    lax.fori_loop(0, _A // 5, bgroup, 0)
    pltpu.sync_copy(outbv_ref, outb_ref.at[pl.ds(base * _NB, _A * _NB)])


def kernel(points, features, batch, W, b):
    n, d = features.shape
    wr = (W * _RADIUS).astype(jnp.bfloat16)
    br = (b * _RADIUS).reshape(1, _NB * 3)

    mesh = plsc.VectorSubcoreMesh(core_axis_name="c", subcore_axis_name="s")
    out_batch = functools.partial(
        pl.kernel,
        out_type=jax.ShapeDtypeStruct((n * _NB,), batch.dtype),
        mesh=mesh,
        compiler_params=pltpu.CompilerParams(needs_layout_passes=False),
        scratch_types=[
            pltpu.VMEM((_STAGE,), jnp.int32),
            pltpu.VMEM((_A * _NB,), jnp.int32),
        ],
    )(_sc_batch_body)(batch)

    rows = pl.pallas_call(
        _tc_body,
        grid=(n // _BLOCK,),
        in_specs=[
            pl.BlockSpec((_BLOCK, d), lambda i: (i, 0)),
            pl.BlockSpec((_BLOCK, 3), lambda i: (i, 0)),
            pl.BlockSpec((d, _NB * 3), lambda i: (0, 0)),
            pl.BlockSpec((1, _NB * 3), lambda i: (0, 0)),
        ],
        out_specs=pl.BlockSpec((_BLOCK, 128), lambda i: (i, 0)),
        out_shape=jax.ShapeDtypeStruct((n, 128), jnp.float32),
    )(features, points, wr, br)

    xt = functools.partial(
        pl.kernel,
        out_type=jax.ShapeDtypeStruct((3, n * _NB), jnp.float32),
        mesh=mesh,
        compiler_params=pltpu.CompilerParams(needs_layout_passes=False),
        scratch_types=[
            pltpu.VMEM((_CH, 128), jnp.float32),
            pltpu.VMEM((_CH, 128), jnp.float32),
            pltpu.VMEM((3, _CH * _NB), jnp.float32),
            pltpu.SemaphoreType.DMA,
            pltpu.SemaphoreType.DMA,
        ],
    )(_sc_points_body)(rows)

    return xt.T, out_batch
